# baseline (speedup 1.0000x reference)
import functools

import jax
import jax.numpy as jnp
from jax import lax
from jax.experimental import pallas as pl
from jax.experimental.pallas import tpu as pltpu
from jax.experimental.pallas import tpu_sc as plsc

_NDIM = 6
_H = 10
_LANES = 16
_NUM_WORKERS = 32
_IDX_CHUNK = 128


def _make_sc_kernel(batch, n_states, out_dim):
    b_per_w = batch // _NUM_WORKERS
    n_idx_chunks = b_per_w // _IDX_CHUNK
    mesh = plsc.VectorSubcoreMesh(core_axis_name="c", subcore_axis_name="s")

    @functools.partial(
        pl.kernel,
        mesh=mesh,
        compiler_params=pltpu.CompilerParams(use_tc_tiling_on_sc=False),
        out_type=jax.ShapeDtypeStruct((batch, out_dim), jnp.float32),
        scratch_types=[
            pltpu.VMEM((_NDIM, b_per_w), jnp.float32),
            pltpu.VMEM((n_idx_chunks, _IDX_CHUNK), jnp.int32),
            pltpu.VMEM((b_per_w, out_dim), jnp.float32),
            pltpu.SemaphoreType.DMA,
        ],
    )
    def sc_kernel(states_hbm, table_hbm, out_hbm, states_v, idx_v, rows_v, sem):
        wid = lax.axis_index("s") * 2 + lax.axis_index("c")
        base = wid * b_per_w
        pltpu.sync_copy(states_hbm.at[:, pl.ds(base, b_per_w)], states_v)

        for i in range(b_per_w // _LANES):
            acc = jnp.zeros((_LANES,), jnp.int32)
            power = 1
            for d in range(_NDIM):
                v = states_v[d, pl.ds(i * _LANES, _LANES)]
                c = jnp.minimum((v * float(_H)).astype(jnp.int32), _H - 1)
                acc = acc + c * power
                power *= _H
            chunk, off = divmod(i * _LANES, _IDX_CHUNK)
            idx_v[chunk, pl.ds(off, _LANES)] = acc

        copies = []
        for c in range(n_idx_chunks):
            cp = pltpu.make_async_copy(
                table_hbm.at[idx_v.at[c]],
                rows_v.at[pl.ds(c * _IDX_CHUNK, _IDX_CHUNK)],
                sem,
            )
            cp.start()
            copies.append(cp)
        for cp in copies:
            cp.wait()
        pltpu.sync_copy(rows_v, out_hbm.at[pl.ds(base, b_per_w)])

    return sc_kernel


def kernel(preprocessed_states, table):
    batch = preprocessed_states.shape[0]
    n_states, out_dim = table.shape
    sc = _make_sc_kernel(batch, n_states, out_dim)
    return sc(preprocessed_states.T, table)
